# Initial kernel scaffold; baseline (speedup 1.0000x reference)
import jax
import jax.numpy as jnp
from jax.experimental import pallas as pl


def kernel(x, emb_table, fc1_w, fc1_b):
    raise NotImplementedError("write your pallas kernel here")


# speedup vs baseline: 9.1974x; 9.1974x over previous
import functools

import jax
import jax.numpy as jnp
from jax import lax
from jax.experimental import pallas as pl
from jax.experimental.pallas import tpu as pltpu
from jax.experimental.pallas import tpu_sc as plsc

NC = 2
NS = 16
LANES = 16
NW = NC * NS

EMB = 32
NCLS = 2
WB_COLS = 48

TOT = 16384 * 20
PER_W = TOT // NW
CHUNK = 128
CPW = PER_W // CHUNK
NBUF = 2
GROUPS = CHUNK // LANES


def _body(x_ref, tab_ref, wb_ref, out_ref,
          idx_v, rows0, rows1, out_v, wb_v, sem0, sem1):
    wid = lax.axis_index("s") * NC + lax.axis_index("c")
    pltpu.sync_copy(x_ref.at[wid], idx_v)
    pltpu.sync_copy(wb_ref, wb_v)

    rows_bufs = (rows0, rows1)
    sems = (sem0, sem1)

    wrow0 = [wb_v[0, pl.ds(j * LANES, LANES)] for j in range(3)]
    wrow1 = [wb_v[1, pl.ds(j * LANES, LANES)] for j in range(3)]
    w0 = [wrow0[k // LANES][k % LANES] for k in range(EMB)]
    w1 = [wrow1[k // LANES][k % LANES] for k in range(EMB)]
    b0 = wrow0[2][0]
    b1 = wrow1[2][0]

    iota = lax.iota(jnp.int32, LANES)
    row_vecs = [iota + g * LANES for g in range(GROUPS)]
    zeros16 = jnp.zeros((LANES,), jnp.int32)
    ones16 = jnp.ones((LANES,), jnp.int32)

    for s in range(NBUF):
        pltpu.async_copy(tab_ref.at[idx_v.at[s]], rows_bufs[s], sems[s])

    @pl.loop(0, CPW // NBUF)
    def _outer(jo):
        for s in range(NBUF):
            c = jo * NBUF + s
            pltpu.make_async_copy(
                tab_ref.at[idx_v.at[s]], rows_bufs[s], sems[s]).wait()
            rows = rows_bufs[s]
            cbase = c * CHUNK
            for g in range(GROUPS):
                acc0 = jnp.zeros((LANES,), jnp.float32)
                acc1 = jnp.zeros((LANES,), jnp.float32)
                for k in range(EMB):
                    kv = jnp.full((LANES,), k, jnp.int32)
                    v = plsc.load_gather(rows, [row_vecs[g], kv])
                    acc0 = acc0 + v * w0[k]
                    acc1 = acc1 + v * w1[k]
                ivec = row_vecs[g] + cbase
                plsc.store_scatter(out_v, [ivec, zeros16], acc0 + b0)
                plsc.store_scatter(out_v, [ivec, ones16], acc1 + b1)
            nxt = c + NBUF

            @pl.when(nxt < CPW)
            def _start_next():
                pltpu.async_copy(
                    tab_ref.at[idx_v.at[nxt]], rows_bufs[s], sems[s])

    pltpu.sync_copy(out_v, out_ref.at[pl.ds(wid * PER_W, PER_W)])


@functools.cache
def _build():
    mesh = plsc.VectorSubcoreMesh(
        core_axis_name="c", subcore_axis_name="s",
        num_cores=NC, num_subcores=NS)
    return pl.kernel(
        _body,
        out_type=jax.ShapeDtypeStruct((TOT, NCLS), jnp.float32),
        mesh=mesh,
        compiler_params=pltpu.CompilerParams(
            needs_layout_passes=False, use_tc_tiling_on_sc=False),
        scratch_types=[
            pltpu.VMEM((CPW, CHUNK), jnp.int32),
            pltpu.VMEM((CHUNK, EMB), jnp.float32),
            pltpu.VMEM((CHUNK, EMB), jnp.float32),
            pltpu.VMEM((PER_W, NCLS), jnp.float32),
            pltpu.VMEM((NCLS, WB_COLS), jnp.float32),
            pltpu.SemaphoreType.DMA,
            pltpu.SemaphoreType.DMA,
        ],
    )


def kernel(x, emb_table, fc1_w, fc1_b):
    x_r = x.reshape(NW, CPW, CHUNK).astype(jnp.int32)
    wb = jnp.zeros((NCLS, WB_COLS), jnp.float32)
    wb = wb.at[:, :EMB].set(fc1_w).at[:, EMB].set(fc1_b)
    return _build()(x_r, emb_table, wb)


# speedup vs baseline: 41.8412x; 4.5492x over previous
import functools

import jax
import jax.numpy as jnp
from jax import lax
from jax.experimental import pallas as pl
from jax.experimental.pallas import tpu as pltpu
from jax.experimental.pallas import tpu_sc as plsc

NC = 2
NS = 16
NW = NC * NS

VOCAB = 1_000_000
EMB = 32
NCLS = 2

TOT = 16384 * 20
PER_W = TOT // NW
CHUNK = 128
CPW = PER_W // CHUNK

BLK = 8192
NBLK = 123
VPAD = NBLK * BLK


def _proj_body(tab_ref, wb_ref, p0_ref, p1_ref):
    w = wb_ref[:, :EMB]
    proj = lax.dot_general(
        w, tab_ref[...], (((1,), (0,)), ((), ())),
        preferred_element_type=jnp.float32,
        precision=lax.Precision.HIGHEST)
    p0_ref[...] = proj[0] + wb_ref[0, EMB]
    p1_ref[...] = proj[1] + wb_ref[1, EMB]


@functools.cache
def _build_proj():
    return pl.pallas_call(
        _proj_body,
        grid=(NBLK,),
        in_specs=[
            pl.BlockSpec((EMB, BLK), lambda i: (0, i)),
            pl.BlockSpec((NCLS, 128), lambda i: (0, 0)),
        ],
        out_specs=[
            pl.BlockSpec((BLK,), lambda i: (i,)),
            pl.BlockSpec((BLK,), lambda i: (i,)),
        ],
        out_shape=[
            jax.ShapeDtypeStruct((VPAD,), jnp.float32),
            jax.ShapeDtypeStruct((VPAD,), jnp.float32),
        ],
    )


def _gather_body(x_ref, p0_ref, p1_ref, out_ref, idx_v, o0_v, o1_v, sem0, sem1):
    wid = lax.axis_index("s") * NC + lax.axis_index("c")
    pltpu.sync_copy(x_ref.at[wid], idx_v)

    @pl.loop(0, CPW)
    def _fire(c):
        dst = pl.ds(c * CHUNK, CHUNK)
        pltpu.async_copy(p0_ref.at[idx_v.at[c]], o0_v.at[dst], sem0)
        pltpu.async_copy(p1_ref.at[idx_v.at[c]], o1_v.at[dst], sem1)

    pltpu.make_async_copy(p0_ref.at[pl.ds(0, PER_W)], o0_v, sem0).wait()
    pltpu.make_async_copy(p1_ref.at[pl.ds(0, PER_W)], o1_v, sem1).wait()

    base = pl.ds(wid * PER_W, PER_W)
    pltpu.sync_copy(o0_v, out_ref.at[0, base])
    pltpu.sync_copy(o1_v, out_ref.at[1, base])


@functools.cache
def _build_gather():
    mesh = plsc.VectorSubcoreMesh(
        core_axis_name="c", subcore_axis_name="s",
        num_cores=NC, num_subcores=NS)
    return pl.kernel(
        _gather_body,
        out_type=jax.ShapeDtypeStruct((NCLS, TOT), jnp.float32),
        mesh=mesh,
        compiler_params=pltpu.CompilerParams(
            needs_layout_passes=False, use_tc_tiling_on_sc=False),
        scratch_types=[
            pltpu.VMEM((CPW, CHUNK), jnp.int32),
            pltpu.VMEM((PER_W,), jnp.float32),
            pltpu.VMEM((PER_W,), jnp.float32),
            pltpu.SemaphoreType.DMA,
            pltpu.SemaphoreType.DMA,
        ],
    )


def kernel(x, emb_table, fc1_w, fc1_b):
    x_r = x.reshape(NW, CPW, CHUNK).astype(jnp.int32)
    wb = jnp.zeros((NCLS, 128), jnp.float32)
    wb = wb.at[:, :EMB].set(fc1_w).at[:, EMB].set(fc1_b)
    p0, p1 = _build_proj()(emb_table.T, wb)
    out = _build_gather()(x_r, p0, p1)
    return out.T


# speedup vs baseline: 61.8181x; 1.4774x over previous
import functools

import jax
import jax.numpy as jnp
from jax import lax
from jax.experimental import pallas as pl
from jax.experimental.pallas import tpu as pltpu
from jax.experimental.pallas import tpu_sc as plsc

NC = 2
NS = 16
NW = NC * NS

VOCAB = 1_000_000
EMB = 32
NCLS = 2

TOT = 16384 * 20
PER_W = TOT // NW
CHUNK = 128
CPW = PER_W // CHUNK

BLK = 16384
NBLK = 62
VPAD = NBLK * BLK


def _proj_body(tab_ref, wb_ref, p0_ref, p1_ref):
    w = wb_ref[:, :EMB]
    proj = lax.dot_general(
        w, tab_ref[...], (((1,), (0,)), ((), ())),
        preferred_element_type=jnp.float32)
    p0_ref[...] = proj[0] + wb_ref[0, EMB]
    p1_ref[...] = proj[1] + wb_ref[1, EMB]


@functools.cache
def _build_proj():
    return pl.pallas_call(
        _proj_body,
        grid=(NBLK,),
        in_specs=[
            pl.BlockSpec((EMB, BLK), lambda i: (0, i)),
            pl.BlockSpec((NCLS, 128), lambda i: (0, 0)),
        ],
        out_specs=[
            pl.BlockSpec((BLK,), lambda i: (i,)),
            pl.BlockSpec((BLK,), lambda i: (i,)),
        ],
        out_shape=[
            jax.ShapeDtypeStruct((VPAD,), jnp.float32),
            jax.ShapeDtypeStruct((VPAD,), jnp.float32),
        ],
    )


def _gather_body(x_ref, p0_ref, p1_ref, out_ref, idx_v, o0_v, o1_v, sem0, sem1):
    wid = lax.axis_index("s") * NC + lax.axis_index("c")
    pltpu.sync_copy(x_ref.at[wid], idx_v)

    @pl.loop(0, CPW)
    def _fire(c):
        dst = pl.ds(c * CHUNK, CHUNK)
        pltpu.async_copy(p0_ref.at[idx_v.at[c]], o0_v.at[dst], sem0)
        pltpu.async_copy(p1_ref.at[idx_v.at[c]], o1_v.at[dst], sem1)

    pltpu.make_async_copy(p0_ref.at[pl.ds(0, PER_W)], o0_v, sem0).wait()
    pltpu.make_async_copy(p1_ref.at[pl.ds(0, PER_W)], o1_v, sem1).wait()

    base = pl.ds(wid * PER_W, PER_W)
    pltpu.sync_copy(o0_v, out_ref.at[0, base])
    pltpu.sync_copy(o1_v, out_ref.at[1, base])


@functools.cache
def _build_gather():
    mesh = plsc.VectorSubcoreMesh(
        core_axis_name="c", subcore_axis_name="s",
        num_cores=NC, num_subcores=NS)
    return pl.kernel(
        _gather_body,
        out_type=jax.ShapeDtypeStruct((NCLS, TOT), jnp.float32),
        mesh=mesh,
        compiler_params=pltpu.CompilerParams(
            needs_layout_passes=False, use_tc_tiling_on_sc=False),
        scratch_types=[
            pltpu.VMEM((CPW, CHUNK), jnp.int32),
            pltpu.VMEM((PER_W,), jnp.float32),
            pltpu.VMEM((PER_W,), jnp.float32),
            pltpu.SemaphoreType.DMA,
            pltpu.SemaphoreType.DMA,
        ],
    )


def kernel(x, emb_table, fc1_w, fc1_b):
    x_r = x.reshape(NW, CPW, CHUNK).astype(jnp.int32)
    wb = jnp.zeros((NCLS, 128), jnp.float32)
    wb = wb.at[:, :EMB].set(fc1_w).at[:, EMB].set(fc1_b)
    p0, p1 = _build_proj()(emb_table.T, wb)
    out = _build_gather()(x_r, p0, p1)
    return out.T


# speedup vs baseline: 69.2853x; 1.1208x over previous
import functools

import jax
import jax.numpy as jnp
from jax import lax
from jax.experimental import pallas as pl
from jax.experimental.pallas import tpu as pltpu
from jax.experimental.pallas import tpu_sc as plsc

NC = 2
NS = 16
LANES = 16
NW = NC * NS

VOCAB = 1_000_000
EMB = 32
NCLS = 2

B = 16384
L = 20
TOT = B * L
PER_W = TOT // NW
BPW = B // NW
CHUNK = 128
CPW = PER_W // CHUNK

BLK = 16384
NBLK = 62
VPAD = NBLK * BLK


def _proj_body(tab_ref, wb_ref, p0_ref, p1_ref):
    w = wb_ref[:, :EMB]
    proj = lax.dot_general(
        w, tab_ref[...], (((1,), (0,)), ((), ())),
        preferred_element_type=jnp.float32)
    p0_ref[...] = proj[0] + wb_ref[0, EMB]
    p1_ref[...] = proj[1] + wb_ref[1, EMB]


@functools.cache
def _build_proj():
    return pl.pallas_call(
        _proj_body,
        grid=(NBLK,),
        in_specs=[
            pl.BlockSpec((EMB, BLK), lambda i: (0, i)),
            pl.BlockSpec((NCLS, 128), lambda i: (0, 0)),
        ],
        out_specs=[
            pl.BlockSpec((BLK,), lambda i: (i,)),
            pl.BlockSpec((BLK,), lambda i: (i,)),
        ],
        out_shape=[
            jax.ShapeDtypeStruct((VPAD,), jnp.float32),
            jax.ShapeDtypeStruct((VPAD,), jnp.float32),
        ],
    )


def _gather_body(xt_ref, p0_ref, p1_ref, out_ref, idxl_v, idx_v, oi_v, sem):
    wid = lax.axis_index("s") * NC + lax.axis_index("c")
    pltpu.sync_copy(xt_ref.at[:, pl.ds(wid * BPW, BPW)], idxl_v)

    iota = lax.iota(jnp.int32, LANES)

    @pl.loop(0, BPW // LANES)
    def _permute(dbg):
        src = pl.ds(dbg * LANES, LANES)
        j20 = (dbg * LANES + iota) * L
        for l in range(L):
            j = j20 + l
            plsc.store_scatter(
                idx_v, [lax.shift_right_logical(j, 7), lax.bitwise_and(j, 127)],
                idxl_v[l, src])

    @pl.loop(0, CPW)
    def _fire(c):
        pltpu.async_copy(p0_ref.at[idx_v.at[c]], oi_v.at[2 * c], sem)
        pltpu.async_copy(p1_ref.at[idx_v.at[c]], oi_v.at[2 * c + 1], sem)

    pltpu.make_async_copy(
        out_ref.at[pl.ds(0, 2 * CPW)], oi_v, sem).wait()

    pltpu.sync_copy(oi_v, out_ref.at[pl.ds(wid * 2 * CPW, 2 * CPW)])


@functools.cache
def _build_gather():
    mesh = plsc.VectorSubcoreMesh(
        core_axis_name="c", subcore_axis_name="s",
        num_cores=NC, num_subcores=NS)
    return pl.kernel(
        _gather_body,
        out_type=jax.ShapeDtypeStruct((NW * 2 * CPW, CHUNK), jnp.float32),
        mesh=mesh,
        compiler_params=pltpu.CompilerParams(
            needs_layout_passes=False, use_tc_tiling_on_sc=False),
        scratch_types=[
            pltpu.VMEM((L, BPW), jnp.int32),
            pltpu.VMEM((CPW, CHUNK), jnp.int32),
            pltpu.VMEM((2 * CPW, CHUNK), jnp.float32),
            pltpu.SemaphoreType.DMA,
        ],
    )


def kernel(x, emb_table, fc1_w, fc1_b):
    xt = x.T.astype(jnp.int32)
    wb = jnp.zeros((NCLS, 128), jnp.float32)
    wb = wb.at[:, :EMB].set(fc1_w).at[:, EMB].set(fc1_b)
    p0, p1 = _build_proj()(emb_table.T, wb)
    out = _build_gather()(xt, p0, p1)
    return out.reshape(TOT // CHUNK, NCLS, CHUNK).transpose(0, 2, 1).reshape(TOT, NCLS)


# speedup vs baseline: 85.3096x; 1.2313x over previous
import functools

import jax
import jax.numpy as jnp
from jax import lax
from jax.experimental import pallas as pl
from jax.experimental.pallas import tpu as pltpu
from jax.experimental.pallas import tpu_sc as plsc

NC = 2
NS = 16
LANES = 16
NW = NC * NS

VOCAB = 1_000_000
EMB = 32
NCLS = 2

B = 16384
L = 20
TOT = B * L
PER_W = TOT // NW
BPW = B // NW
CHUNK = 128
CPW = PER_W // CHUNK

BLK = 16384
NPAIR = 31
VPAD = NPAIR * 2 * BLK


def _proj_body(taba_ref, tabb_ref, w_ref, b_ref, p0_ref, p1_ref):
    dn = (((1,), (0,)), ((), ()))
    pa = lax.dot_general(w_ref[...], taba_ref[...], dn,
                         preferred_element_type=jnp.float32)
    pb = lax.dot_general(w_ref[...], tabb_ref[...], dn,
                         preferred_element_type=jnp.float32)
    p0_ref[pl.ds(0, BLK)] = pa[0] + b_ref[0]
    p0_ref[pl.ds(BLK, BLK)] = pb[0] + b_ref[0]
    p1_ref[pl.ds(0, BLK)] = pa[1] + b_ref[1]
    p1_ref[pl.ds(BLK, BLK)] = pb[1] + b_ref[1]


@functools.cache
def _build_proj():
    return pl.pallas_call(
        _proj_body,
        grid=(NPAIR,),
        in_specs=[
            pl.BlockSpec((EMB, BLK), lambda i: (0, 2 * i)),
            pl.BlockSpec((EMB, BLK), lambda i: (0, 2 * i + 1)),
            pl.BlockSpec((NCLS, EMB), lambda i: (0, 0)),
            pl.BlockSpec((NCLS,), lambda i: (0,)),
        ],
        out_specs=[
            pl.BlockSpec((2 * BLK,), lambda i: (i,)),
            pl.BlockSpec((2 * BLK,), lambda i: (i,)),
        ],
        out_shape=[
            jax.ShapeDtypeStruct((VPAD,), jnp.float32),
            jax.ShapeDtypeStruct((VPAD,), jnp.float32),
        ],
    )


def _gather_body(xt_ref, p0_ref, p1_ref, out_ref, idxl_v, idx_v, oi_v, sem):
    wid = lax.axis_index("s") * NC + lax.axis_index("c")
    pltpu.sync_copy(xt_ref.at[:, pl.ds(wid * BPW, BPW)], idxl_v)

    iota = lax.iota(jnp.int32, LANES)

    @pl.loop(0, BPW // LANES)
    def _permute(dbg):
        src = pl.ds(dbg * LANES, LANES)
        j20 = (dbg * LANES + iota) * L
        for l in range(L):
            j = j20 + l
            plsc.store_scatter(
                idx_v, [lax.shift_right_logical(j, 7), lax.bitwise_and(j, 127)],
                idxl_v[l, src])

    @pl.loop(0, CPW)
    def _fire(c):
        pltpu.async_copy(p0_ref.at[idx_v.at[c]], oi_v.at[2 * c], sem)
        pltpu.async_copy(p1_ref.at[idx_v.at[c]], oi_v.at[2 * c + 1], sem)

    pltpu.make_async_copy(
        out_ref.at[pl.ds(0, 2 * CPW)], oi_v, sem).wait()

    pltpu.sync_copy(oi_v, out_ref.at[pl.ds(wid * 2 * CPW, 2 * CPW)])


@functools.cache
def _build_gather():
    mesh = plsc.VectorSubcoreMesh(
        core_axis_name="c", subcore_axis_name="s",
        num_cores=NC, num_subcores=NS)
    return pl.kernel(
        _gather_body,
        out_type=jax.ShapeDtypeStruct((NW * 2 * CPW, CHUNK), jnp.float32),
        mesh=mesh,
        compiler_params=pltpu.CompilerParams(
            needs_layout_passes=False, use_tc_tiling_on_sc=False),
        scratch_types=[
            pltpu.VMEM((L, BPW), jnp.int32),
            pltpu.VMEM((CPW, CHUNK), jnp.int32),
            pltpu.VMEM((2 * CPW, CHUNK), jnp.float32),
            pltpu.SemaphoreType.DMA,
        ],
    )


def kernel(x, emb_table, fc1_w, fc1_b):
    xt = x.T.astype(jnp.int32)
    tabt = emb_table.T
    p0, p1 = _build_proj()(tabt, tabt, fc1_w, fc1_b)
    out = _build_gather()(xt, p0, p1)
    return out.reshape(TOT // CHUNK, NCLS, CHUNK).transpose(0, 2, 1).reshape(TOT, NCLS)


# speedup vs baseline: 85.3881x; 1.0009x over previous
import functools

import jax
import jax.numpy as jnp
from jax import lax
from jax.experimental import pallas as pl
from jax.experimental.pallas import tpu as pltpu
from jax.experimental.pallas import tpu_sc as plsc

NC = 2
NS = 16
LANES = 16
NW = NC * NS

VOCAB = 1_000_000
EMB = 32
NCLS = 2

B = 16384
L = 20
TOT = B * L
PER_W = TOT // NW
BPW = B // NW
CHUNK = 128
CPW = PER_W // CHUNK

BLK = 16384
NSTEP = 31
VPAD = NSTEP * 2 * BLK


def _proj_body(taba_ref, tabb_ref, w_ref, b_ref, p0_ref, p1_ref):
    dn = (((1,), (0,)), ((), ()))
    for s, tref in enumerate((taba_ref, tabb_ref)):
        p = lax.dot_general(w_ref[...], tref[...], dn,
                            preferred_element_type=jnp.float32)
        p0_ref[pl.ds(s * BLK, BLK)] = p[0] + b_ref[0]
        p1_ref[pl.ds(s * BLK, BLK)] = p[1] + b_ref[1]


@functools.cache
def _build_proj():
    return pl.pallas_call(
        _proj_body,
        grid=(NSTEP,),
        in_specs=[
            pl.BlockSpec((EMB, BLK), lambda i: (0, 2 * i)),
            pl.BlockSpec((EMB, BLK), lambda i: (0, 2 * i + 1)),
            pl.BlockSpec((NCLS, EMB), lambda i: (0, 0)),
            pl.BlockSpec((NCLS,), lambda i: (0,)),
        ],
        out_specs=[
            pl.BlockSpec((2 * BLK,), lambda i: (i,)),
            pl.BlockSpec((2 * BLK,), lambda i: (i,)),
        ],
        out_shape=[
            jax.ShapeDtypeStruct((VPAD,), jnp.float32),
            jax.ShapeDtypeStruct((VPAD,), jnp.float32),
        ],
    )


def _gather_body(xt_ref, p0_ref, p1_ref, out_ref, idxl_v, idx_v, oi_v, sem):
    wid = lax.axis_index("s") * NC + lax.axis_index("c")
    pltpu.sync_copy(xt_ref.at[:, pl.ds(wid * BPW, BPW)], idxl_v)

    iota = lax.iota(jnp.int32, LANES)

    @pl.loop(0, BPW // LANES)
    def _permute(dbg):
        src = pl.ds(dbg * LANES, LANES)
        j20 = (dbg * LANES + iota) * L
        for l in range(L):
            j = j20 + l
            plsc.store_scatter(
                idx_v, [lax.shift_right_logical(j, 7), lax.bitwise_and(j, 127)],
                idxl_v[l, src])

    @pl.loop(0, CPW)
    def _fire(c):
        pltpu.async_copy(p0_ref.at[idx_v.at[c]], oi_v.at[2 * c], sem)
        pltpu.async_copy(p1_ref.at[idx_v.at[c]], oi_v.at[2 * c + 1], sem)

    pltpu.make_async_copy(
        out_ref.at[pl.ds(0, 2 * CPW)], oi_v, sem).wait()

    pltpu.sync_copy(oi_v, out_ref.at[pl.ds(wid * 2 * CPW, 2 * CPW)])


@functools.cache
def _build_gather():
    mesh = plsc.VectorSubcoreMesh(
        core_axis_name="c", subcore_axis_name="s",
        num_cores=NC, num_subcores=NS)
    return pl.kernel(
        _gather_body,
        out_type=jax.ShapeDtypeStruct((NW * 2 * CPW, CHUNK), jnp.float32),
        mesh=mesh,
        compiler_params=pltpu.CompilerParams(
            needs_layout_passes=False, use_tc_tiling_on_sc=False),
        scratch_types=[
            pltpu.VMEM((L, BPW), jnp.int32),
            pltpu.VMEM((CPW, CHUNK), jnp.int32),
            pltpu.VMEM((2 * CPW, CHUNK), jnp.float32),
            pltpu.SemaphoreType.DMA,
        ],
    )


def kernel(x, emb_table, fc1_w, fc1_b):
    xt = x.T.astype(jnp.int32)
    tabt = emb_table.T
    p0, p1 = _build_proj()(tabt, tabt, fc1_w, fc1_b)
    out = _build_gather()(xt, p0, p1)
    return out.reshape(TOT // CHUNK, NCLS, CHUNK).transpose(0, 2, 1).reshape(TOT, NCLS)
